# baseline (speedup 1.0000x reference)
import functools

import jax
import jax.numpy as jnp
from jax import lax
from jax.experimental import pallas as pl
from jax.experimental.pallas import tpu as pltpu
from jax.experimental.pallas import tpu_sc as plsc

_PIECES = 16
_NUM_WORKERS = 32
_NBUF = 6


def _make_shuffle(N: int, C: int, L: int, chunk: int,
                  samples_per_worker: int):
    mesh = plsc.VectorSubcoreMesh(core_axis_name="c", subcore_axis_name="s")

    @functools.partial(
        pl.kernel,
        mesh=mesh,
        out_type=jax.ShapeDtypeStruct((N, C, L), jnp.float32),
        scratch_types=[
            pltpu.VMEM((samples_per_worker, _PIECES), jnp.float32),
            pltpu.VMEM((_NBUF, C, chunk), jnp.float32),
            [pltpu.SemaphoreType.DMA] * _NBUF,
            [pltpu.SemaphoreType.DMA] * _NBUF,
        ],
        compiler_params=pltpu.CompilerParams(needs_layout_passes=False),
    )
    def shuffle(x_hbm, scores_hbm, out_hbm, scores_v,
                bufs, gsems, ssems):
        wid = lax.axis_index("s") * 2 + lax.axis_index("c")
        n0 = wid * samples_per_worker
        pltpu.sync_copy(scores_hbm.at[pl.ds(n0, samples_per_worker)],
                        scores_v)
        offsets = []
        for i in range(samples_per_worker):
            iota = lax.iota(jnp.int32, _PIECES)
            _, perm = plsc.sort_key_val(scores_v[i], iota)
            offsets.append(perm * chunk)
        total = samples_per_worker * _PIECES
        gathers = [None] * _NBUF
        scatters = [None] * _NBUF
        for t in range(total):
            i, k = t // _PIECES, t % _PIECES
            n = wid * samples_per_worker + i
            r = t % _NBUF
            if t >= _NBUF:
                for d in scatters[r]:
                    d.wait()
            src = pl.ds(pl.multiple_of(offsets[i][k], chunk), chunk)
            gathers[r] = [
                pltpu.async_copy(
                    x_hbm.at[n, pl.ds(cg * 8, 8), src],
                    bufs.at[r, pl.ds(cg * 8, 8)], gsems[r])
                for cg in range(C // 8)]
            if t >= 1:
                tp = t - 1
                rp = tp % _NBUF
                ip, kp = tp // _PIECES, tp % _PIECES
                for d in gathers[rp]:
                    d.wait()
                npp = wid * samples_per_worker + ip
                dst = pl.ds(kp * chunk, chunk)
                scatters[rp] = [
                    pltpu.async_copy(
                        bufs.at[rp, pl.ds(cg * 8, 8)],
                        out_hbm.at[npp, pl.ds(cg * 8, 8), dst], ssems[rp])
                    for cg in range(C // 8)]
        last = total - 1
        rl = last % _NBUF
        for d in gathers[rl]:
            d.wait()
        npp = wid * samples_per_worker + last // _PIECES
        dst = pl.ds((last % _PIECES) * chunk, chunk)
        scatters[rl] = [
            pltpu.async_copy(
                bufs.at[rl, pl.ds(cg * 8, 8)],
                out_hbm.at[npp, pl.ds(cg * 8, 8), dst], ssems[rl])
            for cg in range(C // 8)]
        for r in range(_NBUF):
            for d in scatters[r]:
                d.wait()

    return shuffle


def kernel(x):
    N, C, L = x.shape
    chunk = L // _PIECES
    samples_per_worker = N // _NUM_WORKERS
    scores = jax.random.uniform(jax.random.key(42), (N, 1, _PIECES),
                                dtype=jnp.float32).reshape(N, _PIECES)
    return _make_shuffle(N, C, L, chunk, samples_per_worker)(x, scores)


# speedup vs baseline: 1.0177x; 1.0177x over previous
import functools

import jax
import jax.numpy as jnp
from jax import lax
from jax.experimental import pallas as pl
from jax.experimental.pallas import tpu as pltpu
from jax.experimental.pallas import tpu_sc as plsc

_PIECES = 16
_NUM_WORKERS = 32
_NBUF = 6


def _make_shuffle(N: int, C: int, L: int, chunk: int,
                  samples_per_worker: int):
    mesh = plsc.VectorSubcoreMesh(core_axis_name="c", subcore_axis_name="s")

    @functools.partial(
        pl.kernel,
        mesh=mesh,
        out_type=jax.ShapeDtypeStruct((N, C, L), jnp.float32),
        scratch_types=[
            pltpu.VMEM((samples_per_worker, _PIECES), jnp.float32),
            pltpu.VMEM((_NBUF, C, chunk), jnp.float32),
            [pltpu.SemaphoreType.DMA] * _NBUF,
            [pltpu.SemaphoreType.DMA] * _NBUF,
        ],
        compiler_params=pltpu.CompilerParams(needs_layout_passes=False),
    )
    def shuffle(x_hbm, scores_hbm, out_hbm, scores_v,
                bufs, gsems, ssems):
        wid = lax.axis_index("s") * 2 + lax.axis_index("c")
        n0 = wid * samples_per_worker
        pltpu.sync_copy(scores_hbm.at[pl.ds(n0, samples_per_worker)],
                        scores_v)
        offsets = []
        for i in range(samples_per_worker):
            iota = lax.iota(jnp.int32, _PIECES)
            _, perm = plsc.sort_key_val(scores_v[i], iota)
            offsets.append(perm * chunk)
        total = samples_per_worker * _PIECES
        gathers = [None] * _NBUF
        scatters = [None] * _NBUF
        for t in range(total):
            i, k = t // _PIECES, t % _PIECES
            n = wid * samples_per_worker + i
            r = t % _NBUF
            if t >= _NBUF:
                scatters[r].wait()
            gathers[r] = pltpu.async_copy(
                x_hbm.at[n, :, pl.ds(pl.multiple_of(offsets[i][k], chunk),
                                     chunk)],
                bufs.at[r], gsems[r])
            if t >= 1:
                tp = t - 1
                rp = tp % _NBUF
                ip, kp = tp // _PIECES, tp % _PIECES
                gathers[rp].wait()
                scatters[rp] = pltpu.async_copy(
                    bufs.at[rp],
                    out_hbm.at[wid * samples_per_worker + ip, :,
                               pl.ds(kp * chunk, chunk)],
                    ssems[rp])
        last = total - 1
        rl = last % _NBUF
        gathers[rl].wait()
        scatters[rl] = pltpu.async_copy(
            bufs.at[rl],
            out_hbm.at[wid * samples_per_worker + last // _PIECES, :,
                       pl.ds((last % _PIECES) * chunk, chunk)],
            ssems[rl])
        for r in range(_NBUF):
            scatters[r].wait()

    return shuffle


def kernel(x):
    N, C, L = x.shape
    chunk = L // _PIECES
    samples_per_worker = N // _NUM_WORKERS
    scores = jax.random.uniform(jax.random.key(42), (N, 1, _PIECES),
                                dtype=jnp.float32).reshape(N, _PIECES)
    return _make_shuffle(N, C, L, chunk, samples_per_worker)(x, scores)
